# initial kernel scaffold (unmeasured)
import jax
import jax.numpy as jnp
from jax import lax
from jax.experimental import pallas as pl
from jax.experimental.pallas import tpu as pltpu


def kernel(
    x,
):
    def body(*refs):
        pass

    out_shape = jax.ShapeDtypeStruct(..., jnp.float32)
    return pl.pallas_call(body, out_shape=out_shape)(...)


# baseline (device time: 335791 ns/iter reference)
import functools

import jax
import jax.numpy as jnp
from jax import lax
from jax.experimental import pallas as pl
from jax.experimental.pallas import tpu as pltpu

N_Y = 4
M, N = 8192, 1024
CH = M // N_Y


def kernel(x):
    x_bf = x.astype(jnp.bfloat16)

    def body(x_ref, out_ref, comm_rs, comm_ag,
             send_rs, recv_rs, send_ag, recv_ag):
        my_x = lax.axis_index("x")
        my_y = lax.axis_index("y")
        my_z = lax.axis_index("z")
        left = (my_y - 1) % N_Y
        right = (my_y + 1) % N_Y

        barrier_sem = pltpu.get_barrier_semaphore()
        for nbr in [left, right]:
            pl.semaphore_signal(
                barrier_sem, inc=1,
                device_id=(my_x, nbr, my_z),
                device_id_type=pl.DeviceIdType.MESH,
            )
        pl.semaphore_wait(barrier_sem, 2)

        def chunk(c):
            return x_ref[pl.ds(c * CH, CH), :]

        comm_rs[0, :, :] = chunk(my_y)
        for s in range(N_Y - 1):
            send_slot = s % 2
            recv_slot = (s + 1) % 2
            rdma = pltpu.make_async_remote_copy(
                src_ref=comm_rs.at[send_slot],
                dst_ref=comm_rs.at[recv_slot],
                send_sem=send_rs.at[send_slot],
                recv_sem=recv_rs.at[recv_slot],
                device_id=(my_x, right, my_z),
                device_id_type=pl.DeviceIdType.MESH,
            )
            rdma.start()
            rdma.wait()
            c = (my_y - s - 1) % N_Y
            comm_rs[recv_slot, :, :] = comm_rs[recv_slot, :, :] + chunk(c)

        r = (my_y + 1) % N_Y
        out_ref[pl.ds(r * CH, CH), :] = comm_rs[1, :, :]

        comm_ag[0, :, :] = comm_rs[1, :, :]
        for g in range(N_Y - 1):
            send_slot = g % 2
            recv_slot = (g + 1) % 2
            rdma = pltpu.make_async_remote_copy(
                src_ref=comm_ag.at[send_slot],
                dst_ref=comm_ag.at[recv_slot],
                send_sem=send_ag.at[send_slot],
                recv_sem=recv_ag.at[recv_slot],
                device_id=(my_x, right, my_z),
                device_id_type=pl.DeviceIdType.MESH,
            )
            rdma.start()
            rdma.wait()
            origin = (my_y - g) % N_Y
            out_ref[pl.ds(origin * CH, CH), :] = comm_ag[recv_slot, :, :]

        @functools.partial(
            pl.run_scoped, second_barrier=pltpu.SemaphoreType.REGULAR
        )
        def _(second_barrier):
            for nbr in [left, right]:
                pl.semaphore_signal(
                    second_barrier, inc=1,
                    device_id=(my_x, nbr, my_z),
                    device_id_type=pl.DeviceIdType.MESH,
                )
            pl.semaphore_wait(second_barrier, 2)

    out = pl.pallas_call(
        body,
        out_shape=jax.ShapeDtypeStruct((M, N), jnp.bfloat16),
        in_specs=[pl.BlockSpec(memory_space=pltpu.VMEM)],
        out_specs=pl.BlockSpec(memory_space=pltpu.VMEM),
        scratch_shapes=[
            pltpu.VMEM((2, CH, N), jnp.bfloat16),
            pltpu.VMEM((2, CH, N), jnp.bfloat16),
            pltpu.SemaphoreType.DMA((2,)),
            pltpu.SemaphoreType.DMA((2,)),
            pltpu.SemaphoreType.DMA((2,)),
            pltpu.SemaphoreType.DMA((2,)),
        ],
        compiler_params=pltpu.CompilerParams(
            collective_id=0,
            vmem_limit_bytes=100 * 1024 * 1024,
        ),
    )(x_bf)
    return out.astype(jnp.float32)


# device time: 275795 ns/iter; 1.2175x vs baseline; 1.2175x over previous
import functools

import jax
import jax.numpy as jnp
from jax import lax
from jax.experimental import pallas as pl
from jax.experimental.pallas import tpu as pltpu

N_Y = 4
N_Z = 4
M, N = 8192, 1024
SLAB = M // N_Z
PIECE = SLAB // N_Y


def kernel(x):
    x_bf = x.astype(jnp.bfloat16)

    def body(x_ref, out_ref, comm_rs, comm_ag, slab, comm_z,
             send_rs, recv_rs, send_ag, recv_ag, send_z, recv_z):
        my_x = lax.axis_index("x")
        my_y = lax.axis_index("y")
        my_z = lax.axis_index("z")
        y_left = (my_y - 1) % N_Y
        y_right = (my_y + 1) % N_Y
        z_left = (my_z - 1) % N_Z
        z_right = (my_z + 1) % N_Z

        barrier_sem = pltpu.get_barrier_semaphore()
        for dev in [(my_x, y_left, my_z), (my_x, y_right, my_z),
                    (my_x, my_y, z_left), (my_x, my_y, z_right)]:
            pl.semaphore_signal(
                barrier_sem, inc=1,
                device_id=dev, device_id_type=pl.DeviceIdType.MESH,
            )
        pl.semaphore_wait(barrier_sem, 4)

        z0 = my_z * SLAB

        def chunk(c):
            return x_ref[pl.ds(z0 + c * PIECE, PIECE), :]

        comm_rs[0, :, :] = chunk(my_y)
        for s in range(N_Y - 1):
            send_slot = s % 2
            recv_slot = (s + 1) % 2
            rdma = pltpu.make_async_remote_copy(
                src_ref=comm_rs.at[send_slot],
                dst_ref=comm_rs.at[recv_slot],
                send_sem=send_rs.at[send_slot],
                recv_sem=recv_rs.at[recv_slot],
                device_id=(my_x, y_right, my_z),
                device_id_type=pl.DeviceIdType.MESH,
            )
            rdma.start()
            rdma.wait()
            c = (my_y - s - 1) % N_Y
            comm_rs[recv_slot, :, :] = comm_rs[recv_slot, :, :] + chunk(c)

        r = (my_y + 1) % N_Y
        slab[pl.ds(r * PIECE, PIECE), :] = comm_rs[1, :, :]

        comm_ag[0, :, :] = comm_rs[1, :, :]
        for g in range(N_Y - 1):
            send_slot = g % 2
            recv_slot = (g + 1) % 2
            rdma = pltpu.make_async_remote_copy(
                src_ref=comm_ag.at[send_slot],
                dst_ref=comm_ag.at[recv_slot],
                send_sem=send_ag.at[send_slot],
                recv_sem=recv_ag.at[recv_slot],
                device_id=(my_x, y_right, my_z),
                device_id_type=pl.DeviceIdType.MESH,
            )
            rdma.start()
            rdma.wait()
            origin = (my_y - g) % N_Y
            slab[pl.ds(origin * PIECE, PIECE), :] = comm_ag[recv_slot, :, :]

        out_ref[pl.ds(z0, SLAB), :] = slab[:, :]

        for g in range(N_Z - 1):
            recv_slot = (g + 1) % 2
            rdma = pltpu.make_async_remote_copy(
                src_ref=slab if g == 0 else comm_z.at[g % 2],
                dst_ref=comm_z.at[recv_slot],
                send_sem=send_z.at[g % 2],
                recv_sem=recv_z.at[recv_slot],
                device_id=(my_x, my_y, z_right),
                device_id_type=pl.DeviceIdType.MESH,
            )
            rdma.start()
            rdma.wait()
            origin_z = (my_z - g - 1) % N_Z
            out_ref[pl.ds(origin_z * SLAB, SLAB), :] = comm_z[recv_slot, :, :]

        @functools.partial(
            pl.run_scoped, second_barrier=pltpu.SemaphoreType.REGULAR
        )
        def _(second_barrier):
            for dev in [(my_x, y_left, my_z), (my_x, y_right, my_z),
                        (my_x, my_y, z_left), (my_x, my_y, z_right)]:
                pl.semaphore_signal(
                    second_barrier, inc=1,
                    device_id=dev, device_id_type=pl.DeviceIdType.MESH,
                )
            pl.semaphore_wait(second_barrier, 4)

    out = pl.pallas_call(
        body,
        out_shape=jax.ShapeDtypeStruct((M, N), jnp.bfloat16),
        in_specs=[pl.BlockSpec(memory_space=pltpu.VMEM)],
        out_specs=pl.BlockSpec(memory_space=pltpu.VMEM),
        scratch_shapes=[
            pltpu.VMEM((2, PIECE, N), jnp.bfloat16),
            pltpu.VMEM((2, PIECE, N), jnp.bfloat16),
            pltpu.VMEM((SLAB, N), jnp.bfloat16),
            pltpu.VMEM((2, SLAB, N), jnp.bfloat16),
            pltpu.SemaphoreType.DMA((2,)),
            pltpu.SemaphoreType.DMA((2,)),
            pltpu.SemaphoreType.DMA((2,)),
            pltpu.SemaphoreType.DMA((2,)),
            pltpu.SemaphoreType.DMA((2,)),
            pltpu.SemaphoreType.DMA((2,)),
        ],
        compiler_params=pltpu.CompilerParams(
            collective_id=0,
            vmem_limit_bytes=100 * 1024 * 1024,
        ),
    )(x_bf)
    return out.astype(jnp.float32)


# device time: 181233 ns/iter; 1.8528x vs baseline; 1.5218x over previous
import functools

import jax
import jax.numpy as jnp
from jax import lax
from jax.experimental import pallas as pl
from jax.experimental.pallas import tpu as pltpu

N_Y = 4
N_Z = 4
M, N = 8192, 1024
SLAB = 1024
PIECE = SLAB // N_Y


def kernel(x):
    my_x_o = lax.axis_index("x")
    my_z_o = lax.axis_index("z")
    g0 = my_x_o * (N_Z * SLAB) + my_z_o * SLAB
    x_slab = lax.dynamic_slice(x, (g0, 0), (SLAB, N)).astype(jnp.bfloat16)

    def body(x_ref, out_ref, comm_rs, comm_ag, slab_buf, comm_z,
             zslab_store, x_comm,
             send_rs, recv_rs, send_ag, recv_ag, send_z, recv_z,
             send_x, recv_x):
        my_x = lax.axis_index("x")
        my_y = lax.axis_index("y")
        my_z = lax.axis_index("z")
        y_left = (my_y - 1) % N_Y
        y_right = (my_y + 1) % N_Y
        z_left = (my_z - 1) % N_Z
        z_right = (my_z + 1) % N_Z
        other_x = 1 - my_x
        my_half = my_x * (N_Z * SLAB)
        their_half = other_x * (N_Z * SLAB)

        neighbors = [
            (my_x, y_left, my_z), (my_x, y_right, my_z),
            (my_x, my_y, z_left), (my_x, my_y, z_right),
            (other_x, my_y, my_z),
        ]
        barrier_sem = pltpu.get_barrier_semaphore()
        for dev in neighbors:
            pl.semaphore_signal(
                barrier_sem, inc=1,
                device_id=dev, device_id_type=pl.DeviceIdType.MESH,
            )
        pl.semaphore_wait(barrier_sem, len(neighbors))

        def chunk(c):
            return x_ref[pl.ds(c * PIECE, PIECE), :]

        comm_rs[0, :, :] = chunk(my_y)
        for s in range(N_Y - 1):
            rdma = pltpu.make_async_remote_copy(
                src_ref=comm_rs.at[s % 2],
                dst_ref=comm_rs.at[(s + 1) % 2],
                send_sem=send_rs.at[s % 2],
                recv_sem=recv_rs.at[(s + 1) % 2],
                device_id=(my_x, y_right, my_z),
                device_id_type=pl.DeviceIdType.MESH,
            )
            rdma.start()
            rdma.wait()
            c = (my_y - s - 1) % N_Y
            comm_rs[(s + 1) % 2, :, :] = comm_rs[(s + 1) % 2, :, :] + chunk(c)

        r = (my_y + 1) % N_Y
        slab_buf[pl.ds(r * PIECE, PIECE), :] = comm_rs[1, :, :]

        comm_ag[0, :, :] = comm_rs[1, :, :]
        for g in range(N_Y - 1):
            rdma = pltpu.make_async_remote_copy(
                src_ref=comm_ag.at[g % 2],
                dst_ref=comm_ag.at[(g + 1) % 2],
                send_sem=send_ag.at[g % 2],
                recv_sem=recv_ag.at[(g + 1) % 2],
                device_id=(my_x, y_right, my_z),
                device_id_type=pl.DeviceIdType.MESH,
            )
            rdma.start()
            rdma.wait()
            origin = (my_y - g) % N_Y
            slab_buf[pl.ds(origin * PIECE, PIECE), :] = comm_ag[(g + 1) % 2, :, :]

        out_ref[pl.ds(my_half + my_z * SLAB, SLAB), :] = slab_buf[:, :]

        def x_swap(k, src):
            rdma = pltpu.make_async_remote_copy(
                src_ref=src,
                dst_ref=x_comm.at[k % 2],
                send_sem=send_x.at[k % 2],
                recv_sem=recv_x.at[k % 2],
                device_id=(other_x, my_y, my_z),
                device_id_type=pl.DeviceIdType.MESH,
            )
            rdma.start()
            return rdma

        def z_hop(h):
            rdma = pltpu.make_async_remote_copy(
                src_ref=slab_buf if h == 0 else comm_z.at[h % 2],
                dst_ref=comm_z.at[(h + 1) % 2],
                send_sem=send_z.at[h % 2],
                recv_sem=recv_z.at[(h + 1) % 2],
                device_id=(my_x, my_y, z_right),
                device_id_type=pl.DeviceIdType.MESH,
            )
            rdma.start()
            return rdma

        sw = x_swap(0, slab_buf)
        zh = z_hop(0)
        for h in range(N_Z - 1):
            zh.wait()
            zslab_store[h, :, :] = comm_z[(h + 1) % 2, :, :]
            if h < N_Z - 2:
                zh = z_hop(h + 1)
            sw.wait_recv()
            out_ref[pl.ds(their_half + ((my_z - h) % N_Z) * SLAB, SLAB), :] = (
                x_comm[h % 2, :, :]
            )
            sw.wait_send()
            sw = x_swap(h + 1, zslab_store.at[h])
            origin_z = (my_z - h - 1) % N_Z
            out_ref[pl.ds(my_half + origin_z * SLAB, SLAB), :] = (
                zslab_store[h, :, :]
            )

        sw.wait_recv()
        out_ref[pl.ds(their_half + ((my_z - 3) % N_Z) * SLAB, SLAB), :] = (
            x_comm[3 % 2, :, :]
        )
        sw.wait_send()

        @functools.partial(
            pl.run_scoped, second_barrier=pltpu.SemaphoreType.REGULAR
        )
        def _(second_barrier):
            for dev in neighbors:
                pl.semaphore_signal(
                    second_barrier, inc=1,
                    device_id=dev, device_id_type=pl.DeviceIdType.MESH,
                )
            pl.semaphore_wait(second_barrier, len(neighbors))

    out = pl.pallas_call(
        body,
        out_shape=jax.ShapeDtypeStruct((M, N), jnp.bfloat16),
        in_specs=[pl.BlockSpec(memory_space=pltpu.VMEM)],
        out_specs=pl.BlockSpec(memory_space=pltpu.VMEM),
        scratch_shapes=[
            pltpu.VMEM((2, PIECE, N), jnp.bfloat16),
            pltpu.VMEM((2, PIECE, N), jnp.bfloat16),
            pltpu.VMEM((SLAB, N), jnp.bfloat16),
            pltpu.VMEM((2, SLAB, N), jnp.bfloat16),
            pltpu.VMEM((N_Z - 1, SLAB, N), jnp.bfloat16),
            pltpu.VMEM((2, SLAB, N), jnp.bfloat16),
            pltpu.SemaphoreType.DMA((2,)),
            pltpu.SemaphoreType.DMA((2,)),
            pltpu.SemaphoreType.DMA((2,)),
            pltpu.SemaphoreType.DMA((2,)),
            pltpu.SemaphoreType.DMA((2,)),
            pltpu.SemaphoreType.DMA((2,)),
            pltpu.SemaphoreType.DMA((2,)),
            pltpu.SemaphoreType.DMA((2,)),
        ],
        compiler_params=pltpu.CompilerParams(
            collective_id=0,
            vmem_limit_bytes=100 * 1024 * 1024,
        ),
    )(x_slab)
    return out.astype(jnp.float32)
